# baseline (speedup 1.0000x reference)
import functools

import jax
import jax.numpy as jnp
from jax import lax
from jax.experimental import pallas as pl
from jax.experimental.pallas import tpu as pltpu
from jax.experimental.pallas import tpu_sc as plsc

NC, NS, LANES = 2, 16, 16
NW = NC * NS
CH = 16


def _sc_add(x_hbm, w_hbm, out_hbm, bufx, bufw, sinx, sinw, sout):
    R, D = x_hbm.shape
    Lw = w_hbm.shape[0]
    r_per_w = R // NW
    nchunk = r_per_w // CH
    wid = lax.axis_index("s") * NC + lax.axis_index("c")
    base = wid * r_per_w
    wbase = lax.rem(base, Lw)

    def start_in(p, c):
        o = c * CH
        pltpu.async_copy(x_hbm.at[pl.ds(base + o, CH)], bufx.at[p], sinx[p])
        pltpu.async_copy(w_hbm.at[pl.ds(wbase + o, CH)], bufw.at[p], sinw[p])

    def wait_in(p, c):
        o = c * CH
        pltpu.make_async_copy(x_hbm.at[pl.ds(base + o, CH)], bufx.at[p], sinx[p]).wait()
        pltpu.make_async_copy(w_hbm.at[pl.ds(wbase + o, CH)], bufw.at[p], sinw[p]).wait()

    def start_out(p, c):
        o = c * CH
        pltpu.async_copy(bufw.at[p], out_hbm.at[pl.ds(base + o, CH)], sout[p])

    def wait_out(p, c):
        o = c * CH
        pltpu.make_async_copy(bufw.at[p], out_hbm.at[pl.ds(base + o, CH)], sout[p]).wait()

    cols = D // LANES

    start_in(0, 0)
    for c in range(nchunk):
        p = c % 2
        if c + 1 < nchunk:
            if c >= 1:
                wait_out(1 - p, c - 1)
            start_in(1 - p, c + 1)
        wait_in(p, c)

        @plsc.parallel_loop(0, CH * cols, 1, unroll=8)
        def add_body(i):
            r = lax.shift_right_logical(i, 6)
            col = pl.multiple_of(
                lax.shift_left(lax.bitwise_and(i, cols - 1), 4), LANES
            )
            plsc.addupdate(
                bufw.at[p].at[r, pl.ds(col, LANES)],
                bufx[p, r, pl.ds(col, LANES)],
            )

        start_out(p, c)
    wait_out(nchunk % 2, nchunk - 2)
    wait_out(1 - nchunk % 2, nchunk - 1)


def kernel(x, embed_weight):
    B, L, D = x.shape
    mesh = plsc.VectorSubcoreMesh(core_axis_name="c", subcore_axis_name="s")
    sc_call = functools.partial(
        pl.kernel,
        mesh=mesh,
        out_type=jax.ShapeDtypeStruct((B * L, D), jnp.float32),
        scratch_types=[
            pltpu.VMEM((2, CH, D), jnp.float32),
            pltpu.VMEM((2, CH, D), jnp.float32),
            [pltpu.SemaphoreType.DMA] * 2,
            [pltpu.SemaphoreType.DMA] * 2,
            [pltpu.SemaphoreType.DMA] * 2,
        ],
    )(_sc_add)
    out = sc_call(x.reshape(B * L, D), embed_weight)
    return out.reshape(B, L, D)


# speedup vs baseline: 1.3149x; 1.3149x over previous
import functools

import jax
import jax.numpy as jnp
from jax import lax
from jax.experimental import pallas as pl
from jax.experimental.pallas import tpu as pltpu
from jax.experimental.pallas import tpu_sc as plsc

NC, NS, LANES = 2, 16, 16
NW = NC * NS
CH = 16
NXBUF = 4


def _sc_add(x_hbm, w_hbm, out_hbm, bufx, bufw, sinx, sinw, sout):
    R, D = x_hbm.shape
    Lw = w_hbm.shape[0]
    B = R // Lw
    rows_w = Lw // NW
    nchunk = rows_w // CH
    nstep = nchunk * B
    cols = D // LANES
    wid = lax.axis_index("s") * NC + lax.axis_index("c")
    w0 = wid * rows_w

    def x_row(s):
        k, b = divmod(s, B)
        return b * Lw + w0 + k * CH

    def start_x(s):
        p = s % NXBUF
        pltpu.async_copy(x_hbm.at[pl.ds(x_row(s), CH)], bufx.at[p], sinx[p])

    def wait_x(s):
        p = s % NXBUF
        pltpu.make_async_copy(x_hbm.at[pl.ds(x_row(s), CH)], bufx.at[p], sinx[p]).wait()

    def start_w(k):
        q = k % 2
        pltpu.async_copy(w_hbm.at[pl.ds(w0 + k * CH, CH)], bufw.at[q], sinw[q])

    def wait_w(k):
        q = k % 2
        pltpu.make_async_copy(w_hbm.at[pl.ds(w0 + k * CH, CH)], bufw.at[q], sinw[q]).wait()

    def start_out(s):
        p = s % NXBUF
        pltpu.async_copy(bufx.at[p], out_hbm.at[pl.ds(x_row(s), CH)], sout[p])

    def wait_out(s):
        p = s % NXBUF
        pltpu.make_async_copy(bufx.at[p], out_hbm.at[pl.ds(x_row(s), CH)], sout[p]).wait()

    start_w(0)
    start_x(0)
    start_x(1)
    for s in range(nstep):
        k, b = divmod(s, B)
        if b == 0:
            if k + 1 < nchunk:
                start_w(k + 1)
            wait_w(k)
        if s + 2 < nstep:
            if s >= 2:
                wait_out(s - 2)
            start_x(s + 2)
        wait_x(s)
        p = s % NXBUF
        q = k % 2

        @plsc.parallel_loop(0, CH * cols, 1, unroll=8)
        def add_body(i):
            r = lax.shift_right_logical(i, 6)
            col = pl.multiple_of(
                lax.shift_left(lax.bitwise_and(i, cols - 1), 4), LANES
            )
            plsc.addupdate(
                bufx.at[p].at[r, pl.ds(col, LANES)],
                bufw[q, r, pl.ds(col, LANES)],
            )

        start_out(s)
    for s in range(nstep - 4, nstep):
        wait_out(s)


def kernel(x, embed_weight):
    B, L, D = x.shape
    mesh = plsc.VectorSubcoreMesh(core_axis_name="c", subcore_axis_name="s")
    sc_call = functools.partial(
        pl.kernel,
        mesh=mesh,
        out_type=jax.ShapeDtypeStruct((B * L, D), jnp.float32),
        scratch_types=[
            pltpu.VMEM((NXBUF, CH, D), jnp.float32),
            pltpu.VMEM((2, CH, D), jnp.float32),
            [pltpu.SemaphoreType.DMA] * NXBUF,
            [pltpu.SemaphoreType.DMA] * 2,
            [pltpu.SemaphoreType.DMA] * NXBUF,
        ],
    )(_sc_add)
    out = sc_call(x.reshape(B * L, D), embed_weight)
    return out.reshape(B, L, D)


# speedup vs baseline: 1.3184x; 1.0027x over previous
import functools

import jax
import jax.numpy as jnp
from jax import lax
from jax.experimental import pallas as pl
from jax.experimental.pallas import tpu as pltpu
from jax.experimental.pallas import tpu_sc as plsc

NC, NS, LANES = 2, 16, 16
NW = NC * NS
CH = 16
NXBUF = 5


def _sc_add(x_hbm, w_hbm, out_hbm, bufx, bufw, sinx, sinw, sout):
    R, D = x_hbm.shape
    Lw = w_hbm.shape[0]
    B = R // Lw
    rows_w = Lw // NW
    nchunk = rows_w // CH
    nstep = nchunk * B
    cols = D // LANES
    wid = lax.axis_index("s") * NC + lax.axis_index("c")
    w0 = wid * rows_w

    def x_row(s):
        k, b = divmod(s, B)
        return b * Lw + w0 + k * CH

    def start_x(s):
        p = s % NXBUF
        pltpu.async_copy(x_hbm.at[pl.ds(x_row(s), CH)], bufx.at[p], sinx[p])

    def wait_x(s):
        p = s % NXBUF
        pltpu.make_async_copy(x_hbm.at[pl.ds(x_row(s), CH)], bufx.at[p], sinx[p]).wait()

    def start_w(k):
        q = k % 2
        pltpu.async_copy(w_hbm.at[pl.ds(w0 + k * CH, CH)], bufw.at[q], sinw[q])

    def wait_w(k):
        q = k % 2
        pltpu.make_async_copy(w_hbm.at[pl.ds(w0 + k * CH, CH)], bufw.at[q], sinw[q]).wait()

    def start_out(s):
        p = s % NXBUF
        pltpu.async_copy(bufx.at[p], out_hbm.at[pl.ds(x_row(s), CH)], sout[p])

    def wait_out(s):
        p = s % NXBUF
        pltpu.make_async_copy(bufx.at[p], out_hbm.at[pl.ds(x_row(s), CH)], sout[p]).wait()

    start_w(0)
    start_x(0)
    start_x(1)
    start_x(2)
    for s in range(nstep):
        k, b = divmod(s, B)
        if b == 0:
            if k + 1 < nchunk:
                start_w(k + 1)
            wait_w(k)
        if s + 3 < nstep:
            if s >= 2:
                wait_out(s - 2)
            start_x(s + 3)
        wait_x(s)
        p = s % NXBUF
        q = k % 2

        @plsc.parallel_loop(0, CH * cols, 1, unroll=8)
        def add_body(i):
            r = lax.shift_right_logical(i, 6)
            col = pl.multiple_of(
                lax.shift_left(lax.bitwise_and(i, cols - 1), 4), LANES
            )
            plsc.addupdate(
                bufx.at[p].at[r, pl.ds(col, LANES)],
                bufw[q, r, pl.ds(col, LANES)],
            )

        start_out(s)
    for s in range(nstep - 5, nstep):
        wait_out(s)


def kernel(x, embed_weight):
    B, L, D = x.shape
    mesh = plsc.VectorSubcoreMesh(core_axis_name="c", subcore_axis_name="s")
    sc_call = functools.partial(
        pl.kernel,
        mesh=mesh,
        out_type=jax.ShapeDtypeStruct((B * L, D), jnp.float32),
        scratch_types=[
            pltpu.VMEM((NXBUF, CH, D), jnp.float32),
            pltpu.VMEM((2, CH, D), jnp.float32),
            [pltpu.SemaphoreType.DMA] * NXBUF,
            [pltpu.SemaphoreType.DMA] * 2,
            [pltpu.SemaphoreType.DMA] * NXBUF,
        ],
    )(_sc_add)
    out = sc_call(x.reshape(B * L, D), embed_weight)
    return out.reshape(B, L, D)


# speedup vs baseline: 1.3188x; 1.0003x over previous
import functools

import jax
import jax.numpy as jnp
from jax import lax
from jax.experimental import pallas as pl
from jax.experimental.pallas import tpu as pltpu
from jax.experimental.pallas import tpu_sc as plsc

NC, NS, LANES = 2, 16, 16
NW = NC * NS
CH = 16
NXBUF = 5


def _sc_add(x_hbm, w_hbm, out_hbm, bufx, bufw, sinx, sinw, sout):
    R, D = x_hbm.shape
    Lw = w_hbm.shape[0]
    B = R // Lw
    rows_w = Lw // NW
    nchunk = rows_w // CH
    nstep = nchunk * B
    cols = D // LANES
    wid = lax.axis_index("s") * NC + lax.axis_index("c")
    w0 = wid * rows_w

    def x_row(s):
        k, b = divmod(s, B)
        return b * Lw + w0 + k * CH

    def start_x(s):
        p = s % NXBUF
        pltpu.async_copy(x_hbm.at[pl.ds(x_row(s), CH)], bufx.at[p], sinx[p])

    def wait_x(s):
        p = s % NXBUF
        pltpu.make_async_copy(x_hbm.at[pl.ds(x_row(s), CH)], bufx.at[p], sinx[p]).wait()

    def start_w(k):
        q = k % 2
        pltpu.async_copy(w_hbm.at[pl.ds(w0 + k * CH, CH)], bufw.at[q], sinw[q])

    def wait_w(k):
        q = k % 2
        pltpu.make_async_copy(w_hbm.at[pl.ds(w0 + k * CH, CH)], bufw.at[q], sinw[q]).wait()

    def start_out(s):
        p = s % NXBUF
        pltpu.async_copy(bufx.at[p], out_hbm.at[pl.ds(x_row(s), CH)], sout[p])

    def wait_out(s):
        p = s % NXBUF
        pltpu.make_async_copy(bufx.at[p], out_hbm.at[pl.ds(x_row(s), CH)], sout[p]).wait()

    start_w(0)
    start_x(0)
    start_x(1)
    start_x(2)
    for s in range(nstep):
        k, b = divmod(s, B)
        if b == 0:
            if k + 1 < nchunk:
                start_w(k + 1)
            wait_w(k)
        if s + 3 < nstep:
            if s >= 2:
                wait_out(s - 2)
            start_x(s + 3)
        wait_x(s)
        p = s % NXBUF
        q = k % 2

        cshift = cols.bit_length() - 1

        @plsc.parallel_loop(0, CH * cols, 1, unroll=8)
        def add_body(i):
            r = lax.shift_right_logical(i, cshift)
            col = pl.multiple_of(
                lax.shift_left(lax.bitwise_and(i, cols - 1), 4), LANES
            )
            plsc.addupdate(
                bufx.at[p].at[r, pl.ds(col, LANES)],
                bufw[q, r, pl.ds(col, LANES)],
            )

        start_out(s)
    for s in range(nstep - 5, nstep):
        wait_out(s)


def kernel(x, embed_weight):
    B, L, D = x.shape
    mesh = plsc.VectorSubcoreMesh(core_axis_name="c", subcore_axis_name="s")
    sc_call = functools.partial(
        pl.kernel,
        mesh=mesh,
        out_type=jax.ShapeDtypeStruct((B * L, D), jnp.float32),
        scratch_types=[
            pltpu.VMEM((NXBUF, CH, D), jnp.float32),
            pltpu.VMEM((2, CH, D), jnp.float32),
            [pltpu.SemaphoreType.DMA] * NXBUF,
            [pltpu.SemaphoreType.DMA] * 2,
            [pltpu.SemaphoreType.DMA] * NXBUF,
        ],
    )(_sc_add)
    out = sc_call(x.reshape(B * L, D), embed_weight)
    return out.reshape(B, L, D)
